# baseline (speedup 1.0000x reference)
import functools

import jax
import jax.numpy as jnp
from jax import lax
from jax.experimental import pallas as pl
from jax.experimental.pallas import tpu as pltpu
from jax.experimental.pallas import tpu_sc as plsc

_LANES = 16
_NUM_WORKERS = 32


@functools.lru_cache(maxsize=None)
def _make_sc_kernel(batch: int, dim: int):
    b_per_w = batch // _NUM_WORKERS
    assert batch % (_NUM_WORKERS * _LANES) == 0
    assert dim == 2 * _LANES

    mesh = plsc.VectorSubcoreMesh(core_axis_name="c", subcore_axis_name="s")

    @functools.partial(
        pl.kernel,
        mesh=mesh,
        compiler_params=pltpu.CompilerParams(
            needs_layout_passes=False, use_tc_tiling_on_sc=False),
        out_type=jax.ShapeDtypeStruct((batch,), jnp.float32),
        scratch_types=[
            pltpu.VMEM((b_per_w,), jnp.int32),
            pltpu.VMEM((b_per_w,), jnp.int32),
            pltpu.VMEM((b_per_w, dim), jnp.float32),
            pltpu.VMEM((b_per_w, dim), jnp.float32),
            pltpu.VMEM((b_per_w,), jnp.float32),
            pltpu.VMEM((b_per_w,), jnp.float32),
            pltpu.VMEM((b_per_w,), jnp.float32),
            pltpu.SemaphoreType.DMA,
        ],
    )
    def k(uidx_hbm, midx_hbm, uemb_hbm, memb_hbm, ub_hbm, mb_hbm, out_hbm,
          uidx_v, midx_v, urows_v, mrows_v, ub_v, mb_v, out_v, sem):
        wid = lax.axis_index("s") * 2 + lax.axis_index("c")
        base = wid * b_per_w
        pltpu.sync_copy(uidx_hbm.at[pl.ds(base, b_per_w)], uidx_v)
        pltpu.sync_copy(midx_hbm.at[pl.ds(base, b_per_w)], midx_v)
        cu = pltpu.async_copy(uemb_hbm.at[uidx_v], urows_v, sem)
        cm = pltpu.async_copy(memb_hbm.at[midx_v], mrows_v, sem)
        cub = pltpu.async_copy(ub_hbm.at[uidx_v], ub_v, sem)
        cmb = pltpu.async_copy(mb_hbm.at[midx_v], mb_v, sem)
        cu.wait()
        cm.wait()
        cub.wait()
        cmb.wait()

        zeros = jnp.zeros((_LANES,), jnp.int32)

        def group(g, carry):
            b0 = g * _LANES
            out_v[pl.ds(b0, _LANES)] = (ub_v[pl.ds(b0, _LANES)]
                                        + mb_v[pl.ds(b0, _LANES)])
            for r in range(_LANES):
                b = b0 + r
                t = (urows_v[b, pl.ds(0, _LANES)] * mrows_v[b, pl.ds(0, _LANES)]
                     + urows_v[b, pl.ds(_LANES, _LANES)]
                     * mrows_v[b, pl.ds(_LANES, _LANES)])
                plsc.addupdate_scatter(out_v, [zeros + b], t)
            return carry

        lax.fori_loop(0, b_per_w // _LANES, group, 0)
        pltpu.sync_copy(out_v, out_hbm.at[pl.ds(base, b_per_w)])

    return k


def kernel(user_indices, movie_indices, user_emb, movie_emb, user_bias, movie_bias):
    batch = user_indices.shape[0]
    dim = user_emb.shape[1]
    k = _make_sc_kernel(batch, dim)
    return k(
        user_indices.astype(jnp.int32),
        movie_indices.astype(jnp.int32),
        user_emb,
        movie_emb,
        user_bias.reshape(-1),
        movie_bias.reshape(-1),
    )
